# Initial kernel scaffold; baseline (speedup 1.0000x reference)
import jax
import jax.numpy as jnp
from jax.experimental import pallas as pl


def kernel(x, neighbour_index, W1, b1, W2, b2):
    raise NotImplementedError("write your pallas kernel here")


# speedup vs baseline: 2.2328x; 2.2328x over previous
import functools

import jax
import jax.numpy as jnp
from jax import lax
from jax.experimental import pallas as pl
from jax.experimental.pallas import tpu as pltpu
from jax.experimental.pallas import tpu_sc as plsc

N = 100000
D_TOTAL = 128
D_DYN = 32
D_ANC = 96
HIDDEN = 64
NSTEPS = 4
DT = 0.25

NC = 2
NS = 16
NW = NC * NS
ROWS = 3 * N
R_PER_W = ROWS // NW
CHUNK = 125
NCHUNK = R_PER_W // CHUNK
GRP = 5
NGRP = NCHUNK // GRP

_HIGH = lax.Precision.HIGHEST


@functools.lru_cache(maxsize=None)
def _make_gather(width):
  mesh = plsc.VectorSubcoreMesh(core_axis_name="c", subcore_axis_name="s")

  @functools.partial(
      pl.kernel,
      out_type=jax.ShapeDtypeStruct((NW * NCHUNK, CHUNK, width), jnp.float32),
      mesh=mesh,
      compiler_params=pltpu.CompilerParams(use_tc_tiling_on_sc=False),
      scratch_types=[
          pltpu.VMEM((NCHUNK, CHUNK), jnp.int32),
          pltpu.VMEM((GRP, CHUNK, width), jnp.float32),
          pltpu.SemaphoreType.DMA,
      ],
  )
  def gather_kernel(idx_hbm, table_hbm, out_hbm, idx_v, buf, sem):
    wid = lax.axis_index("s") * NC + lax.axis_index("c")
    pltpu.sync_copy(idx_hbm.at[wid], idx_v)

    def body(g, carry):
      copies = [
          pltpu.async_copy(table_hbm.at[idx_v.at[g * GRP + b]], buf.at[b], sem)
          for b in range(GRP)
      ]
      for cp in copies:
        cp.wait()
      pltpu.sync_copy(buf, out_hbm.at[pl.ds(wid * NCHUNK + g * GRP, GRP)])
      return carry

    lax.fori_loop(0, NGRP, body, 0)

  return gather_kernel


_BLK = 2000
_NBLK = N // _BLK


def _pre_body(anc_ref, ganc_ref, wa_ref, wn_ref, b1_ref, out_ref):
  out_ref[...] = (
      b1_ref[...]
      + jnp.dot(anc_ref[...], wa_ref[...], precision=_HIGH)
      + jnp.dot(ganc_ref[...], wn_ref[...], precision=_HIGH)
  )


def _step_body(dyn_ref, g_ref, a_ref, wd_ref, wn_ref, w2_ref, b2_ref, out_ref):
  h = (
      a_ref[...]
      + jnp.dot(dyn_ref[...], wd_ref[...], precision=_HIGH)
      + jnp.dot(g_ref[...], wn_ref[...], precision=_HIGH)
  )
  fz = jnp.dot(jax.nn.gelu(h), w2_ref[...], precision=_HIGH) + b2_ref[...]
  out_ref[...] = dyn_ref[...] + DT * fz


def _row_spec(w):
  return pl.BlockSpec((_BLK, w), lambda i: (i, 0))


def _full_spec(r, c):
  return pl.BlockSpec((r, c), lambda i: (0, 0))


_precompute = pl.pallas_call(
    _pre_body,
    grid=(_NBLK,),
    in_specs=[
        _row_spec(D_ANC),
        _row_spec(3 * D_ANC),
        _full_spec(D_ANC, HIDDEN),
        _full_spec(3 * D_ANC, HIDDEN),
        _full_spec(1, HIDDEN),
    ],
    out_specs=_row_spec(HIDDEN),
    out_shape=jax.ShapeDtypeStruct((N, HIDDEN), jnp.float32),
)

_step = pl.pallas_call(
    _step_body,
    grid=(_NBLK,),
    in_specs=[
        _row_spec(D_DYN),
        _row_spec(3 * D_DYN),
        _row_spec(HIDDEN),
        _full_spec(D_DYN, HIDDEN),
        _full_spec(3 * D_DYN, HIDDEN),
        _full_spec(HIDDEN, D_DYN),
        _full_spec(1, D_DYN),
    ],
    out_specs=_row_spec(D_DYN),
    out_shape=jax.ShapeDtypeStruct((N, D_DYN), jnp.float32),
)


def kernel(x, neighbour_index, W1, b1, W2, b2):
  idx3 = neighbour_index[:, 1:].reshape(NW, NCHUNK, CHUNK)
  dyn = x[:, :D_DYN]
  anc = x[:, D_DYN:]

  w1r = W1.reshape(4, D_TOTAL, HIDDEN)
  wd_self = w1r[0, :D_DYN]
  wd_nbr = w1r[1:, :D_DYN].reshape(3 * D_DYN, HIDDEN)
  wa_self = w1r[0, D_DYN:]
  wa_nbr = w1r[1:, D_DYN:].reshape(3 * D_ANC, HIDDEN)

  ganc = _make_gather(D_ANC)(idx3, anc).reshape(N, 3 * D_ANC)
  a_const = _precompute(anc, ganc, wa_self, wa_nbr, b1.reshape(1, HIDDEN))

  for _ in range(NSTEPS):
    g = _make_gather(D_DYN)(idx3, dyn).reshape(N, 3 * D_DYN)
    dyn = _step(dyn, g, a_const, wd_self, wd_nbr, W2, b2.reshape(1, D_DYN))

  return jnp.concatenate([dyn, anc], axis=1)


# speedup vs baseline: 2.7764x; 1.2434x over previous
import functools

import jax
import jax.numpy as jnp
from jax import lax
from jax.experimental import pallas as pl
from jax.experimental.pallas import tpu as pltpu
from jax.experimental.pallas import tpu_sc as plsc

N = 100000
D_TOTAL = 128
D_DYN = 32
D_ANC = 96
HIDDEN = 64
NSTEPS = 4
DT = 0.25

NC = 2
NS = 16
NW = NC * NS
P_PER_W = N // NW
CHUNK = 125
NCH = P_PER_W // CHUNK

_HIGH = lax.Precision.HIGHEST


@functools.lru_cache(maxsize=None)
def _make_gather(nj, width):
  mesh = plsc.VectorSubcoreMesh(core_axis_name="c", subcore_axis_name="s")

  @functools.partial(
      pl.kernel,
      out_type=jax.ShapeDtypeStruct((N, nj * width), jnp.float32),
      mesh=mesh,
      compiler_params=pltpu.CompilerParams(use_tc_tiling_on_sc=False),
      scratch_types=[
          pltpu.VMEM((NCH, nj, CHUNK), jnp.int32),
          pltpu.VMEM((nj, CHUNK, width), jnp.float32),
          pltpu.SemaphoreType.DMA,
      ],
  )
  def gather_kernel(idx_hbm, table_hbm, out_hbm, idx_v, buf, sem):
    wid = lax.axis_index("s") * NC + lax.axis_index("c")
    pltpu.sync_copy(idx_hbm.at[wid], idx_v)

    def body(c, carry):
      copies = [
          pltpu.async_copy(table_hbm.at[idx_v.at[c, j]], buf.at[j], sem)
          for j in range(nj)
      ]
      for cp in copies:
        cp.wait()
      base = wid * P_PER_W + c * CHUNK
      for j in range(nj):
        pltpu.sync_copy(
            buf.at[j],
            out_hbm.at[pl.ds(base, CHUNK), pl.ds(j * width, width)],
        )
      return carry

    lax.fori_loop(0, NCH, body, 0)

  return gather_kernel


_BLK = 4000
_NBLK = N // _BLK


def _pre_body(anc_ref, ganc_ref, wa_ref, wn_ref, b1_ref, out_ref):
  out_ref[...] = (
      b1_ref[...]
      + jnp.dot(anc_ref[...], wa_ref[...], precision=_HIGH)
      + jnp.dot(ganc_ref[...], wn_ref[...], precision=_HIGH)
  )


def _step_body(g4_ref, a_ref, wd_ref, w2_ref, b2_ref, out_ref):
  g4 = g4_ref[...]
  h = a_ref[...] + jnp.dot(g4, wd_ref[...], precision=_HIGH)
  fz = jnp.dot(jax.nn.gelu(h), w2_ref[...], precision=_HIGH) + b2_ref[...]
  out_ref[...] = g4[:, :D_DYN] + DT * fz


def _row_spec(w):
  return pl.BlockSpec((_BLK, w), lambda i: (i, 0))


def _full_spec(r, c):
  return pl.BlockSpec((r, c), lambda i: (0, 0))


_precompute = pl.pallas_call(
    _pre_body,
    grid=(_NBLK,),
    in_specs=[
        _row_spec(D_ANC),
        _row_spec(3 * D_ANC),
        _full_spec(D_ANC, HIDDEN),
        _full_spec(3 * D_ANC, HIDDEN),
        _full_spec(1, HIDDEN),
    ],
    out_specs=_row_spec(HIDDEN),
    out_shape=jax.ShapeDtypeStruct((N, HIDDEN), jnp.float32),
)

_step = pl.pallas_call(
    _step_body,
    grid=(_NBLK,),
    in_specs=[
        _row_spec(4 * D_DYN),
        _row_spec(HIDDEN),
        _full_spec(4 * D_DYN, HIDDEN),
        _full_spec(HIDDEN, D_DYN),
        _full_spec(1, D_DYN),
    ],
    out_specs=_row_spec(D_DYN),
    out_shape=jax.ShapeDtypeStruct((N, D_DYN), jnp.float32),
)


def kernel(x, neighbour_index, W1, b1, W2, b2):
  nb = neighbour_index.reshape(NW, NCH, CHUNK, 4)
  idx4 = nb.transpose(0, 1, 3, 2)
  idx3 = nb[..., 1:].transpose(0, 1, 3, 2)
  dyn = x[:, :D_DYN]
  anc = x[:, D_DYN:]

  w1r = W1.reshape(4, D_TOTAL, HIDDEN)
  wd = w1r[:, :D_DYN].reshape(4 * D_DYN, HIDDEN)
  wa_self = w1r[0, D_DYN:]
  wa_nbr = w1r[1:, D_DYN:].reshape(3 * D_ANC, HIDDEN)

  ganc = _make_gather(3, D_ANC)(idx3, anc)
  a_const = _precompute(anc, ganc, wa_self, wa_nbr, b1.reshape(1, HIDDEN))

  table = dyn
  for _ in range(NSTEPS):
    g4 = _make_gather(4, D_DYN)(idx4, table)
    table = _step(g4, a_const, wd, W2, b2.reshape(1, D_DYN))

  return jnp.concatenate([table, anc], axis=1)
